# baseline (speedup 1.0000x reference)
import functools

import jax
import jax.numpy as jnp
from jax import lax
from jax.experimental import pallas as pl
from jax.experimental.pallas import tpu as pltpu
from jax.experimental.pallas import tpu_sc as plsc

D = 64
DP = 128
BATCH = 4096
HIST = 200
B = BATCH * HIST
NC = 2
NS = 16
NW = NC * NS
ROWS_PER_W = BATCH // NW
B_PER_W = B // NW
N_OUTER = ROWS_PER_W // 2
UNROLL = 8

_mesh = plsc.VectorSubcoreMesh(core_axis_name="c", subcore_axis_name="s")


@functools.partial(
    pl.kernel,
    mesh=_mesh,
    out_type=jax.ShapeDtypeStruct((BATCH, HIST, D), jnp.float32),
    scratch_types=[
        pltpu.VMEM((B_PER_W,), jnp.int32),
        pltpu.VMEM((HIST, DP), jnp.float32),
        pltpu.VMEM((HIST, DP), jnp.float32),
        pltpu.VMEM((HIST, D), jnp.float32),
        pltpu.VMEM((HIST, D), jnp.float32),
        pltpu.SemaphoreType.DMA,
        pltpu.SemaphoreType.DMA,
        pltpu.SemaphoreType.DMA,
        pltpu.SemaphoreType.DMA,
    ],
    compiler_params=pltpu.CompilerParams(use_tc_tiling_on_sc=True),
)
def _gather_kernel(idx_hbm, table_hbm, out_hbm,
                   idx_all, x0, x1, y0, y1, sg0, sg1, ss0, ss1):
    wid = lax.axis_index("s") * NC + lax.axis_index("c")
    base_w = wid * ROWS_PER_W

    pltpu.sync_copy(idx_hbm.at[pl.ds(base_w * HIST, B_PER_W)], idx_all)

    def start_gather(j, x, sem):
        pltpu.async_copy(table_hbm.at[idx_all.at[pl.ds(j * HIST, HIST)]],
                         x, sem)

    def wait_gather(x, sem):
        pltpu.make_async_copy(table_hbm.at[idx_all.at[pl.ds(0, HIST)]],
                              x, sem).wait()

    def repack(x, y):
        def rbody(r, carry):
            for u in range(UNROLL):
                for c in range(D // 16):
                    y[r * UNROLL + u, pl.ds(c * 16, 16)] = (
                        x[r * UNROLL + u, pl.ds(c * 16, 16)])
            return carry
        lax.fori_loop(0, HIST // UNROLL, rbody, 0)

    def start_store(j, y, sem):
        pltpu.async_copy(y, out_hbm.at[base_w + j], sem)

    def wait_store(y, sem):
        pltpu.make_async_copy(y, out_hbm.at[0], sem).wait()

    start_gather(0, x0, sg0)
    start_gather(1, x1, sg1)
    wait_gather(x0, sg0)
    repack(x0, y0)
    start_store(0, y0, ss0)
    start_gather(2, x0, sg0)
    wait_gather(x1, sg1)
    repack(x1, y1)
    start_store(1, y1, ss1)

    def body(i, carry):
        j = 2 * i
        start_gather(j + 1, x1, sg1)
        wait_gather(x0, sg0)
        wait_store(y0, ss0)
        repack(x0, y0)
        start_store(j, y0, ss0)
        start_gather(j + 2, x0, sg0)
        wait_gather(x1, sg1)
        wait_store(y1, ss1)
        repack(x1, y1)
        start_store(j + 1, y1, ss1)
        return carry

    lax.fori_loop(1, N_OUTER - 1, body, 0)

    j = ROWS_PER_W - 2
    start_gather(j + 1, x1, sg1)
    wait_gather(x0, sg0)
    wait_store(y0, ss0)
    repack(x0, y0)
    start_store(j, y0, ss0)
    wait_gather(x1, sg1)
    wait_store(y1, ss1)
    repack(x1, y1)
    start_store(j + 1, y1, ss1)
    wait_store(y0, ss0)
    wait_store(y1, ss1)


def kernel(inputs, word_embedding_matrix):
    idx = inputs.reshape(-1).astype(jnp.int32)
    table_p = jnp.pad(word_embedding_matrix, ((0, 0), (0, DP - D)))
    return _gather_kernel(idx, table_p)
